# Initial kernel scaffold; baseline (speedup 1.0000x reference)
import jax
import jax.numpy as jnp
from jax.experimental import pallas as pl

_LANES = 2048


def _first_spike_krnl(spk_ref, tgt_ref, first_ref, tgt_out_ref):
    T = spk_ref.shape[0]
    s = spk_ref[...]
    tvals = jax.lax.broadcasted_iota(jnp.float32, s.shape, 0)
    cand = jnp.where(s > 0.5, tvals, jnp.float32(T - 1))
    first_ref[...] = jnp.min(cand, axis=0)
    tg = tgt_ref[...]
    tgt_out_ref[...] = jnp.where(tg < 0, tg + T, tg)


def kernel(spk_out, targets):
    T, B, N = spk_out.shape
    BN = B * N
    assert BN % _LANES == 0
    grid = BN // _LANES
    spk2 = spk_out.reshape(T, BN)
    tflat = targets.reshape(BN)

    first, tgt_out = pl.pallas_call(
        _first_spike_krnl,
        grid=(grid,),
        in_specs=[
            pl.BlockSpec((T, _LANES), lambda i: (0, i)),
            pl.BlockSpec((_LANES,), lambda i: (i,)),
        ],
        out_specs=[
            pl.BlockSpec((_LANES,), lambda i: (i,)),
            pl.BlockSpec((_LANES,), lambda i: (i,)),
        ],
        out_shape=[
            jax.ShapeDtypeStruct((BN,), jnp.float32),
            jax.ShapeDtypeStruct((BN,), jnp.float32),
        ],
    )(spk2, tflat)

    return first.reshape(B, N), tgt_out.reshape(B, N)


# speedup vs baseline: 1.3168x; 1.3168x over previous
import jax
import jax.numpy as jnp
from jax.experimental import pallas as pl

_LANES = 2048


def _first_spike_krnl(spk_ref, tgt_ref, first_ref, tgt_out_ref):
    T = spk_ref.shape[0]
    s = spk_ref[...]
    tvals = jax.lax.broadcasted_iota(jnp.int32, s.shape, 0)
    cand = jnp.where(s > 0.5, tvals, jnp.int32(T - 1))
    first_ref[...] = jnp.min(cand, axis=0).astype(jnp.float32)
    tg = tgt_ref[...]
    tgt_out_ref[...] = jnp.where(tg < 0, tg + T, tg)


def kernel(spk_out, targets):
    T, B, N = spk_out.shape
    BN = B * N
    assert BN % _LANES == 0
    grid = BN // _LANES
    spk2 = spk_out.reshape(T, BN)
    tflat = targets.reshape(BN)

    first, tgt_out = pl.pallas_call(
        _first_spike_krnl,
        grid=(grid,),
        in_specs=[
            pl.BlockSpec((T, _LANES), lambda i: (0, i)),
            pl.BlockSpec((_LANES,), lambda i: (i,)),
        ],
        out_specs=[
            pl.BlockSpec((_LANES,), lambda i: (i,)),
            pl.BlockSpec((_LANES,), lambda i: (i,)),
        ],
        out_shape=[
            jax.ShapeDtypeStruct((BN,), jnp.float32),
            jax.ShapeDtypeStruct((BN,), jnp.float32),
        ],
    )(spk2, tflat)

    return first.reshape(B, N), tgt_out.reshape(B, N)


# speedup vs baseline: 1.5448x; 1.1731x over previous
import jax
import jax.numpy as jnp
from jax.experimental import pallas as pl

_LANES = 10240


def _first_spike_krnl(spk_ref, tgt_ref, first_ref, tgt_out_ref):
    T = spk_ref.shape[0]
    s = spk_ref[...]
    tvals = jax.lax.broadcasted_iota(jnp.int32, s.shape, 0)
    cand = jnp.where(s > 0.5, tvals, jnp.int32(T - 1))
    first_ref[...] = jnp.min(cand, axis=0).astype(jnp.float32)
    tg = tgt_ref[...]
    tgt_out_ref[...] = jnp.where(tg < 0, tg + T, tg)


def kernel(spk_out, targets):
    T, B, N = spk_out.shape
    BN = B * N
    assert BN % _LANES == 0
    grid = BN // _LANES
    spk2 = spk_out.reshape(T, BN)
    tflat = targets.reshape(BN)

    first, tgt_out = pl.pallas_call(
        _first_spike_krnl,
        grid=(grid,),
        in_specs=[
            pl.BlockSpec((T, _LANES), lambda i: (0, i)),
            pl.BlockSpec((_LANES,), lambda i: (i,)),
        ],
        out_specs=[
            pl.BlockSpec((_LANES,), lambda i: (i,)),
            pl.BlockSpec((_LANES,), lambda i: (i,)),
        ],
        out_shape=[
            jax.ShapeDtypeStruct((BN,), jnp.float32),
            jax.ShapeDtypeStruct((BN,), jnp.float32),
        ],
    )(spk2, tflat)

    return first.reshape(B, N), tgt_out.reshape(B, N)


# speedup vs baseline: 2.5672x; 1.6619x over previous
import jax
import jax.numpy as jnp
from jax.experimental import pallas as pl

_B_BLK = 16


def _first_spike_krnl(spk_ref, tgt_ref, first_ref, tgt_out_ref):
    T = spk_ref.shape[0]
    s = spk_ref[...]
    tvals = jax.lax.broadcasted_iota(jnp.int32, s.shape, 0)
    cand = jnp.where(s > 0.5, tvals, jnp.int32(T - 1))
    first_ref[...] = jnp.min(cand, axis=0).astype(jnp.float32)
    tg = tgt_ref[...]
    tgt_out_ref[...] = jnp.where(tg < 0, tg + T, tg)


def kernel(spk_out, targets):
    T, B, N = spk_out.shape
    assert B % _B_BLK == 0
    grid = B // _B_BLK

    first, tgt_out = pl.pallas_call(
        _first_spike_krnl,
        grid=(grid,),
        in_specs=[
            pl.BlockSpec((T, _B_BLK, N), lambda i: (0, i, 0)),
            pl.BlockSpec((_B_BLK, N), lambda i: (i, 0)),
        ],
        out_specs=[
            pl.BlockSpec((_B_BLK, N), lambda i: (i, 0)),
            pl.BlockSpec((_B_BLK, N), lambda i: (i, 0)),
        ],
        out_shape=[
            jax.ShapeDtypeStruct((B, N), jnp.float32),
            jax.ShapeDtypeStruct((B, N), jnp.float32),
        ],
    )(spk_out, targets)

    return first, tgt_out
